# baseline (speedup 1.0000x reference)
import functools

import jax
import jax.numpy as jnp
from jax import lax
from jax.experimental import pallas as pl
from jax.experimental.pallas import tpu as pltpu
from jax.experimental.pallas import tpu_sc as plsc

_SQRT_2_OVER_PI = 0.7978845608028654
_EPS = 1e-5

NC = 2
NS = 16
L = 16


def _head_body(x_ref, w_ref, b_ref, g_ref, be_ref, o_ref):
    xb = x_ref[...]
    h = lax.dot_general(xb, w_ref[...], (((1,), (1,)), ((), ())),
                        preferred_element_type=jnp.float32)
    h = h + b_ref[...]
    h = 0.5 * h * (1.0 + jnp.tanh(_SQRT_2_OVER_PI * (h + 0.044715 * h * h * h)))
    mean = jnp.mean(h, axis=-1, keepdims=True)
    var = jnp.mean(jnp.square(h - mean), axis=-1, keepdims=True)
    o_ref[...] = g_ref[...] * (h - mean) * lax.rsqrt(var + _EPS) + be_ref[...]


def _dense_head(x, W, b, gamma, beta):
    B, INP = x.shape
    H = W.shape[0]
    blk = 512
    return pl.pallas_call(
        _head_body,
        grid=(B // blk,),
        in_specs=[
            pl.BlockSpec((blk, INP), lambda i: (i, 0)),
            pl.BlockSpec((H, INP), lambda i: (0, 0)),
            pl.BlockSpec((1, H), lambda i: (0, 0)),
            pl.BlockSpec((1, H), lambda i: (0, 0)),
            pl.BlockSpec((1, H), lambda i: (0, 0)),
        ],
        out_specs=pl.BlockSpec((blk, H), lambda i: (i, 0)),
        out_shape=jax.ShapeDtypeStruct((B, H), jnp.float32),
    )(x, W, b.reshape(1, H), gamma.reshape(1, H), beta.reshape(1, H))


def _make_sc_transpose(V, H):
    CW = 256
    NRING = 3
    NCH = V // CW
    TAIL = V - NCH * CW
    NW = NC * NS
    KMAX = -(-NCH // NW)
    JMAX = -(-KMAX // NRING)

    mesh = plsc.VectorSubcoreMesh(core_axis_name="c", subcore_axis_name="s")

    @functools.partial(
        pl.kernel,
        out_type=jax.ShapeDtypeStruct((V * H,), jnp.float32),
        mesh=mesh,
        compiler_params=pltpu.CompilerParams(needs_layout_passes=False),
        scratch_types=[
            [pltpu.VMEM((H, CW), jnp.float32) for _ in range(NRING)],
            [pltpu.VMEM((CW * H,), jnp.float32) for _ in range(NRING)],
            [pltpu.SemaphoreType.DMA for _ in range(NRING)],
            [pltpu.SemaphoreType.DMA for _ in range(NRING)],
        ],
    )
    def sc_tr(tt_hbm, tail_hbm, out_hbm, ins, outs, sems_i, sems_o):
        cid = lax.axis_index("c")
        sid = lax.axis_index("s")
        wid = sid * NC + cid

        iota = lax.iota(jnp.int32, L)
        obase = [(cb + iota) * H for cb in range(0, CW, L)]
        cols = [cb + iota for cb in range(0, CW, L)]

        def chunk_id(k):
            return k * NW + wid

        def start_in(k, slot):
            pltpu.async_copy(
                tt_hbm.at[pl.ds(0, H), pl.ds(chunk_id(k) * CW, CW)],
                ins[slot], sems_i[slot])

        def wait_in(slot):
            pltpu.make_async_copy(tt_hbm.at[pl.ds(0, H), pl.ds(0, CW)],
                                  ins[slot], sems_i[slot]).wait()

        def wait_out(slot):
            pltpu.make_async_copy(outs[slot],
                                  out_hbm.at[pl.ds(0, CW * H)],
                                  sems_o[slot]).wait()

        def compute(k, slot):
            inb, outf = ins[slot], outs[slot]

            @plsc.parallel_loop(0, H, unroll=2)
            def _(i):
                d = lax.bitwise_and(iota + i, H - 1)
                for blk in range(CW // L):
                    v = plsc.load_gather(inb, [d, cols[blk]])
                    plsc.store_scatter(outf, [obase[blk] + d], v)
            pltpu.async_copy(outf,
                             out_hbm.at[pl.ds(chunk_id(k) * CW * H, CW * H)],
                             sems_o[slot])

        def valid(k):
            return chunk_id(k) < NCH

        def body(j, carry):
            for slot in range(NRING):
                k = NRING * j + slot

                @pl.when(valid(k))
                def _():
                    wait_in(slot)

                    @pl.when(j > 0)
                    def _():
                        wait_out(slot)

                    compute(k, slot)

                    @pl.when(valid(k + NRING))
                    def _():
                        start_in(k + NRING, slot)

            return carry

        for slot in range(NRING):
            @pl.when(valid(slot))
            def _():
                start_in(slot, slot)

        lax.fori_loop(0, JMAX, body, 0)
        for slot in range(NRING):
            @pl.when(valid(slot))
            def _():
                wait_out(slot)

        if TAIL:
            @pl.when(wid == NW - 1)
            def _():
                pltpu.sync_copy(tail_hbm, outs[0].at[pl.ds(0, TAIL * H)])
                pltpu.sync_copy(outs[0].at[pl.ds(0, TAIL * H)],
                                out_hbm.at[pl.ds(NCH * CW * H, TAIL * H)])

    return sc_tr


def _make_sc_dot(B, C, H, V):
    NW = NC * NS
    ROWS = B // NW
    assert C >= L and C % 8 == 0
    offs = list(range(0, C - L + 1, L))
    if offs[-1] + L < C:
        offs.append(C - L)
    NBUF = 4
    assert ROWS % NBUF == 0

    mesh = plsc.VectorSubcoreMesh(core_axis_name="c", subcore_axis_name="s")

    @functools.partial(
        pl.kernel,
        out_type=jax.ShapeDtypeStruct((B * C,), jnp.float32),
        mesh=mesh,
        compiler_params=pltpu.CompilerParams(needs_layout_passes=False,
                                             use_tc_tiling_on_sc=False),
        scratch_types=[
            pltpu.VMEM((ROWS, H), jnp.float32),
            pltpu.VMEM((ROWS * C,), jnp.int32),
            [pltpu.VMEM((C, H), jnp.float32) for _ in range(NBUF)],
            [pltpu.VMEM((C,), jnp.float32) for _ in range(NBUF)],
            pltpu.VMEM((H + L,), jnp.float32),
            pltpu.VMEM((ROWS * C,), jnp.float32),
            [pltpu.SemaphoreType.DMA for _ in range(NBUF)],
            [pltpu.SemaphoreType.DMA for _ in range(NBUF)],
        ],
    )
    def sc_dot(h_hbm, cand_hbm, table_hbm, bias_hbm, out_hbm,
               h_v, cand_v, embs, biases, hext, out_v, sems_e, sems_b):
        cid = lax.axis_index("c")
        sid = lax.axis_index("s")
        wid = sid * NC + cid
        base = wid * ROWS

        pltpu.sync_copy(h_hbm.at[pl.ds(base, ROWS)], h_v)
        pltpu.sync_copy(cand_hbm.at[pl.ds(base * C, ROWS * C)], cand_v)

        iota = lax.iota(jnp.int32, L)
        rows_g = [o + iota for o in offs]

        def start_row(r, j):
            idx = cand_v.at[pl.ds(r * C, C)]
            pltpu.async_copy(table_hbm.at[idx], embs[j], sems_e[j])
            pltpu.async_copy(bias_hbm.at[idx], biases[j], sems_b[j])

        def wait_row(j):
            idx = cand_v.at[pl.ds(0, C)]
            pltpu.make_async_copy(table_hbm.at[idx], embs[j], sems_e[j]).wait()
            pltpu.make_async_copy(bias_hbm.at[idx], biases[j], sems_b[j]).wait()

        def compute_row(r, j):
            for k in range(H // L):
                hext[pl.ds(k * L, L)] = h_v[r, pl.ds(k * L, L)]
            hext[pl.ds(H, L)] = h_v[r, pl.ds(0, L)]
            emb_b, bias_b = embs[j], biases[j]

            def iloop(i, accs):
                t = iota + i
                col = lax.bitwise_and(t, H - 1)
                hwin = plsc.load_gather(hext, [t])
                return tuple(
                    acc + plsc.load_gather(emb_b, [rg, col]) * hwin
                    for acc, rg in zip(accs, rows_g)
                )

            accs = lax.fori_loop(
                0, H, iloop,
                tuple(bias_b[pl.ds(o, L)] for o in offs))
            out_base = r * C
            for acc, o in zip(accs, offs):
                out_v[pl.ds(out_base + o, L)] = acc

        def body(i, carry):
            for j in range(NBUF):
                wait_row(j)
                compute_row(NBUF * i + j, j)

                @pl.when(i < ROWS // NBUF - 1)
                def _():
                    start_row(NBUF * i + j + NBUF, j)

            return carry

        for j in range(NBUF):
            start_row(j, j)
        lax.fori_loop(0, ROWS // NBUF, body, 0)

        pltpu.sync_copy(out_v, out_hbm.at[pl.ds(base * C, ROWS * C)])

    return sc_dot


def kernel(x, candidates, W, b, gamma, beta, table, bias):
    B, C = candidates.shape
    V, H = table.shape
    h = _dense_head(x, W, b, gamma, beta)
    cand_flat = candidates.astype(jnp.int32).reshape(-1)
    bias_flat = bias.reshape(-1)
    CW = 256
    tail = table[(V // CW) * CW:, :].reshape(-1)
    sc_tr = _make_sc_transpose(V, H)
    table_lin = sc_tr(table.T, tail)
    sc_dot = _make_sc_dot(B, C, H, V)
    logits = sc_dot(h, cand_flat, table_lin.reshape(V, H), bias_flat)
    return logits.reshape(B, C)


# speedup vs baseline: 1.2147x; 1.2147x over previous
import functools

import jax
import jax.numpy as jnp
from jax import lax
from jax.experimental import pallas as pl
from jax.experimental.pallas import tpu as pltpu
from jax.experimental.pallas import tpu_sc as plsc

_SQRT_2_OVER_PI = 0.7978845608028654
_EPS = 1e-5

NC = 2
NS = 16
L = 16


def _head_body(x_ref, w_ref, b_ref, g_ref, be_ref, o_ref):
    xb = x_ref[...]
    h = lax.dot_general(xb, w_ref[...], (((1,), (1,)), ((), ())),
                        preferred_element_type=jnp.float32)
    h = h + b_ref[...]
    h = 0.5 * h * (1.0 + jnp.tanh(_SQRT_2_OVER_PI * (h + 0.044715 * h * h * h)))
    mean = jnp.mean(h, axis=-1, keepdims=True)
    var = jnp.mean(jnp.square(h - mean), axis=-1, keepdims=True)
    o_ref[...] = g_ref[...] * (h - mean) * lax.rsqrt(var + _EPS) + be_ref[...]


def _dense_head(x, W, b, gamma, beta):
    B, INP = x.shape
    H = W.shape[0]
    blk = 512
    return pl.pallas_call(
        _head_body,
        grid=(B // blk,),
        in_specs=[
            pl.BlockSpec((blk, INP), lambda i: (i, 0)),
            pl.BlockSpec((H, INP), lambda i: (0, 0)),
            pl.BlockSpec((1, H), lambda i: (0, 0)),
            pl.BlockSpec((1, H), lambda i: (0, 0)),
            pl.BlockSpec((1, H), lambda i: (0, 0)),
        ],
        out_specs=pl.BlockSpec((blk, H), lambda i: (i, 0)),
        out_shape=jax.ShapeDtypeStruct((B, H), jnp.float32),
    )(x, W, b.reshape(1, H), gamma.reshape(1, H), beta.reshape(1, H))


def _make_sc_transpose(V, H):
    CW = 256
    NRING = 4
    HP = H // 2
    NCH = V // CW
    TAIL = V - NCH * CW
    NW = NC * NS
    KMAX = -(-NCH // NW)
    JMAX = -(-KMAX // NRING)

    mesh = plsc.VectorSubcoreMesh(core_axis_name="c", subcore_axis_name="s")

    @functools.partial(
        pl.kernel,
        out_type=jax.ShapeDtypeStruct((V * HP,), jnp.float32),
        mesh=mesh,
        compiler_params=pltpu.CompilerParams(needs_layout_passes=False),
        scratch_types=[
            [pltpu.VMEM((H, CW), jnp.float32) for _ in range(NRING)],
            [pltpu.VMEM((CW * HP,), jnp.float32) for _ in range(NRING)],
            [pltpu.SemaphoreType.DMA for _ in range(NRING)],
            [pltpu.SemaphoreType.DMA for _ in range(NRING)],
        ],
    )
    def sc_tr(tt_hbm, tail_hbm, out_hbm, ins, outs, sems_i, sems_o):
        cid = lax.axis_index("c")
        sid = lax.axis_index("s")
        wid = sid * NC + cid

        iota = lax.iota(jnp.int32, L)
        obase = [(cb + iota) * HP for cb in range(0, CW, L)]
        cols = [cb + iota for cb in range(0, CW, L)]

        def chunk_id(k):
            return k * NW + wid

        def start_in(k, slot):
            pltpu.async_copy(
                tt_hbm.at[pl.ds(0, H), pl.ds(chunk_id(k) * CW, CW)],
                ins[slot], sems_i[slot])

        def wait_in(slot):
            pltpu.make_async_copy(tt_hbm.at[pl.ds(0, H), pl.ds(0, CW)],
                                  ins[slot], sems_i[slot]).wait()

        def wait_out(slot):
            pltpu.make_async_copy(outs[slot],
                                  out_hbm.at[pl.ds(0, CW * HP)],
                                  sems_o[slot]).wait()

        def compute(k, slot):
            inb, outf = ins[slot], outs[slot]

            @plsc.parallel_loop(0, HP, unroll=2)
            def _(i):
                p = lax.bitwise_and(iota + i, HP - 1)
                d0 = p * 2
                for blk in range(CW // L):
                    v0 = plsc.load_gather(inb, [d0, cols[blk]])
                    v1 = plsc.load_gather(inb, [d0 + 1, cols[blk]])
                    pk = plsc.bitcast(
                        plsc.pack(v0, v1,
                                  format=plsc.PackFormat.INTERLEAVED),
                        jnp.float32)
                    plsc.store_scatter(outf, [obase[blk] + p], pk)
            pltpu.async_copy(outf,
                             out_hbm.at[pl.ds(chunk_id(k) * CW * HP, CW * HP)],
                             sems_o[slot])

        def valid(k):
            return chunk_id(k) < NCH

        def body(j, carry):
            for slot in range(NRING):
                k = NRING * j + slot

                @pl.when(valid(k))
                def _():
                    wait_in(slot)

                    @pl.when(j > 0)
                    def _():
                        wait_out(slot)

                    compute(k, slot)

                    @pl.when(valid(k + NRING))
                    def _():
                        start_in(k + NRING, slot)

            return carry

        for slot in range(NRING):
            @pl.when(valid(slot))
            def _():
                start_in(slot, slot)

        lax.fori_loop(0, JMAX, body, 0)
        for slot in range(NRING):
            @pl.when(valid(slot))
            def _():
                wait_out(slot)

        if TAIL:
            @pl.when(wid == NW - 1)
            def _():
                stage = outs[1]
                pck = outs[0]
                pltpu.sync_copy(tail_hbm, stage.at[pl.ds(0, TAIL * H)])

                @plsc.parallel_loop(0, TAIL, unroll=2)
                def _(r):
                    for blk in range(HP // L):
                        kk = blk * L + iota
                        v0 = plsc.load_gather(stage, [r * H + 2 * kk])
                        v1 = plsc.load_gather(stage, [r * H + 2 * kk + 1])
                        pk = plsc.bitcast(
                            plsc.pack(v0, v1,
                                      format=plsc.PackFormat.INTERLEAVED),
                            jnp.float32)
                        plsc.store_scatter(pck, [r * HP + kk], pk)

                pltpu.sync_copy(pck.at[pl.ds(0, TAIL * HP)],
                                out_hbm.at[pl.ds(NCH * CW * HP, TAIL * HP)])

    return sc_tr


def _make_sc_dot(B, C, H, V):
    NW = NC * NS
    ROWS = B // NW
    HP = H // 2
    assert C >= L and C % 8 == 0
    offs = list(range(0, C - L + 1, L))
    if offs[-1] + L < C:
        offs.append(C - L)
    NBUF = 4
    assert ROWS % NBUF == 0

    mesh = plsc.VectorSubcoreMesh(core_axis_name="c", subcore_axis_name="s")

    @functools.partial(
        pl.kernel,
        out_type=jax.ShapeDtypeStruct((B * C,), jnp.float32),
        mesh=mesh,
        compiler_params=pltpu.CompilerParams(needs_layout_passes=False,
                                             use_tc_tiling_on_sc=False),
        scratch_types=[
            pltpu.VMEM((ROWS, H), jnp.float32),
            pltpu.VMEM((ROWS * C,), jnp.int32),
            [pltpu.VMEM((C, HP), jnp.float32) for _ in range(NBUF)],
            [pltpu.VMEM((C,), jnp.float32) for _ in range(NBUF)],
            pltpu.VMEM((H,), jnp.float32),
            pltpu.VMEM((ROWS * C,), jnp.float32),
            [pltpu.SemaphoreType.DMA for _ in range(NBUF)],
            [pltpu.SemaphoreType.DMA for _ in range(NBUF)],
        ],
    )
    def sc_dot(h_hbm, cand_hbm, table_hbm, bias_hbm, out_hbm,
               h_v, cand_v, embs, biases, hext, out_v, sems_e, sems_b):
        cid = lax.axis_index("c")
        sid = lax.axis_index("s")
        wid = sid * NC + cid
        base = wid * ROWS

        pltpu.sync_copy(h_hbm.at[pl.ds(base, ROWS)], h_v)
        pltpu.sync_copy(cand_hbm.at[pl.ds(base * C, ROWS * C)], cand_v)

        iota = lax.iota(jnp.int32, L)
        rows_g = [o + iota for o in offs]

        def start_row(r, j):
            idx = cand_v.at[pl.ds(r * C, C)]
            pltpu.async_copy(table_hbm.at[idx], embs[j], sems_e[j])
            pltpu.async_copy(bias_hbm.at[idx], biases[j], sems_b[j])

        def wait_row(j):
            idx = cand_v.at[pl.ds(0, C)]
            pltpu.make_async_copy(table_hbm.at[idx], embs[j], sems_e[j]).wait()
            pltpu.make_async_copy(bias_hbm.at[idx], biases[j], sems_b[j]).wait()

        def compute_row(r, j):
            for k in range(H // L):
                hext[pl.ds(k * L, L)] = h_v[r, pl.ds(k * L, L)]
            emb_b, bias_b = embs[j], biases[j]

            def iloop(i, accs):
                p = lax.bitwise_and(iota + i, HP - 1)
                d0 = p * 2
                hw0 = plsc.load_gather(hext, [d0])
                hw1 = plsc.load_gather(hext, [d0 + 1])
                new = []
                for acc, rg in zip(accs, rows_g):
                    e = plsc.load_gather(emb_b, [rg, p])
                    va, vb = plsc.unpack(
                        plsc.bitcast(e, jnp.bfloat16),
                        format=plsc.PackFormat.INTERLEAVED)
                    new.append(acc + va * hw0 + vb * hw1)
                return tuple(new)

            accs = lax.fori_loop(
                0, HP, iloop,
                tuple(bias_b[pl.ds(o, L)] for o in offs))
            out_base = r * C
            for acc, o in zip(accs, offs):
                out_v[pl.ds(out_base + o, L)] = acc

        def body(i, carry):
            for j in range(NBUF):
                wait_row(j)
                compute_row(NBUF * i + j, j)

                @pl.when(i < ROWS // NBUF - 1)
                def _():
                    start_row(NBUF * i + j + NBUF, j)

            return carry

        for j in range(NBUF):
            start_row(j, j)
        lax.fori_loop(0, ROWS // NBUF, body, 0)

        pltpu.sync_copy(out_v, out_hbm.at[pl.ds(base * C, ROWS * C)])

    return sc_dot


def kernel(x, candidates, W, b, gamma, beta, table, bias):
    B, C = candidates.shape
    V, H = table.shape
    h = _dense_head(x, W, b, gamma, beta)
    cand_flat = candidates.astype(jnp.int32).reshape(-1)
    bias_flat = bias.reshape(-1)
    CW = 256
    tail = table[(V // CW) * CW:, :].reshape(-1)
    sc_tr = _make_sc_transpose(V, H)
    table_pk = sc_tr(table.T, tail)
    sc_dot = _make_sc_dot(B, C, H, V)
    logits = sc_dot(h, cand_flat, table_pk.reshape(V, H // 2), bias_flat)
    return logits.reshape(B, C)
